# baseline (speedup 1.0000x reference)
import jax
import jax.numpy as jnp
from jax.experimental import pallas as pl

B, S, D = 4, 2048, 2048
H, E = 8, 512
TOPK = 8
EPS = 1e-5

M = B * S
HE = H * E
BM1 = 256
NM1 = M // BM1
BMF = 128
NMF = M // BMF
BM2 = 512
NM2 = M // BM2
SB2 = S // BM2


def _stats_tile(z):
    rows = jax.lax.broadcasted_iota(jnp.int32, (8, 128), 0)
    cols = jax.lax.broadcasted_iota(jnp.int32, (8, 128), 1)
    acc = jnp.zeros((8, 128), jnp.float32)
    for h in range(H):
        zh = z[:, h * E:(h + 1) * E]
        s = jnp.sum(zh)
        q = jnp.sum(zh * zh)
        acc = acc + jnp.where((rows == h) & (cols == 0), s, 0.0)
        acc = acc + jnp.where((rows == h) & (cols == 1), q, 0.0)
    return acc.reshape(1, 8, 128)


def _softmax_tile(z, c, bm):
    z4 = z.reshape(bm, H, E)
    rowmax = jnp.max(z4, axis=2, keepdims=True)
    t = jnp.minimum(rowmax, c)
    e = jnp.where(z4 >= t, jnp.exp(z4 - rowmax), 0.0)
    den = jnp.sum(e, axis=2, keepdims=True)
    return (e / den).reshape(1, bm, H, E)


def _mm_stats_kernel(x_ref, w_ref, z_ref, st_ref):
    z = jax.lax.dot_general(
        x_ref[...], w_ref[...], (((1,), (1,)), ((), ())),
        preferred_element_type=jnp.float32)
    z_ref[...] = z
    st_ref[...] = _stats_tile(z)


def _mm_stats(x2, w):
    return pl.pallas_call(
        _mm_stats_kernel,
        grid=(NM1,),
        in_specs=[
            pl.BlockSpec((BM1, D), lambda m: (m, 0)),
            pl.BlockSpec((HE, D), lambda m: (0, 0)),
        ],
        out_specs=[
            pl.BlockSpec((BM1, HE), lambda m: (m, 0)),
            pl.BlockSpec((1, 8, 128), lambda m: (m, 0, 0)),
        ],
        out_shape=[
            jax.ShapeDtypeStruct((M, HE), jnp.float32),
            jax.ShapeDtypeStruct((NM1, 8, 128), jnp.float32),
        ],
    )(x2, w)


def _thresholds(st):
    s = st[:, :, 0].sum(axis=0)
    q = st[:, :, 1].sum(axis=0)
    n = float(M * E)
    mean = s / n
    var = q / n - mean * mean
    p = 1.0 - float(TOPK) / float(E)
    sigma = jnp.sqrt(2.0) * jax.scipy.special.erfinv(2.0 * p - 1.0)
    c = sigma * jnp.sqrt(var + EPS) + mean
    return jnp.broadcast_to(c[:, None], (H, 128))


def _mm_softmax_kernel(x_ref, w_ref, za_ref, c_ref, zb_ref, st_ref, o_ref):
    z = jax.lax.dot_general(
        x_ref[...], w_ref[...], (((1,), (1,)), ((), ())),
        preferred_element_type=jnp.float32)
    zb_ref[...] = z
    st_ref[...] = _stats_tile(z)
    za = za_ref[...]
    for h in range(H):
        zh = za[:, h * E:(h + 1) * E]
        rowmax = jnp.max(zh, axis=1, keepdims=True)
        t = jnp.minimum(rowmax, c_ref[h, 0])
        e = jnp.where(zh >= t, jnp.exp(zh - rowmax), 0.0)
        den = jnp.sum(e, axis=1, keepdims=True)
        o_ref[:, h * E:(h + 1) * E] = e / den


def _mm_softmax(x2, w, za, c):
    return pl.pallas_call(
        _mm_softmax_kernel,
        grid=(NMF,),
        in_specs=[
            pl.BlockSpec((BMF, D), lambda m: (m, 0)),
            pl.BlockSpec((HE, D), lambda m: (0, 0)),
            pl.BlockSpec((BMF, HE), lambda m: (m, 0)),
            pl.BlockSpec((H, 128), lambda m: (0, 0)),
        ],
        out_specs=[
            pl.BlockSpec((BMF, HE), lambda m: (m, 0)),
            pl.BlockSpec((1, 8, 128), lambda m: (m, 0, 0)),
            pl.BlockSpec((BMF, HE), lambda m: (m, 0)),
        ],
        out_shape=[
            jax.ShapeDtypeStruct((M, HE), jnp.float32),
            jax.ShapeDtypeStruct((NMF, 8, 128), jnp.float32),
            jax.ShapeDtypeStruct((M, HE), jnp.float32),
        ],
        input_output_aliases={2: 2},
    )(x2, w, za, c)


def _softmax_kernel(z_ref, c_ref, o_ref):
    c = c_ref[...][:, :1]
    o_ref[...] = _softmax_tile(z_ref[...], c, BM2)


def _masked_softmax(z, c):
    return pl.pallas_call(
        _softmax_kernel,
        grid=(NM2,),
        in_specs=[
            pl.BlockSpec((BM2, HE), lambda m: (m, 0)),
            pl.BlockSpec((H, 128), lambda m: (0, 0)),
        ],
        out_specs=pl.BlockSpec(
            (1, BM2, H, E), lambda m: (m // SB2, m % SB2, 0, 0)),
        out_shape=jax.ShapeDtypeStruct((B, S, H, E), jnp.float32),
    )(z, c)


def kernel(x, W1, W2):
    x2 = x.reshape(M, D)
    z1, st1 = _mm_stats(x2, W1)
    c1 = _thresholds(st1)
    z2, st2, g1 = _mm_softmax(x2, W2, z1, c1)
    c2 = _thresholds(st2)
    g2 = _masked_softmax(z2, c2)
    return g1.reshape(B, S, H, E), g2


# speedup vs baseline: 1.1340x; 1.1340x over previous
import jax
import jax.numpy as jnp
from jax.experimental import pallas as pl

B, S, D = 4, 2048, 2048
H, E = 8, 512
TOPK = 8
EPS = 1e-5

M = B * S
HE = H * E
BM1 = 256
NM1 = M // BM1
BMF = 128
NMF = M // BMF
BM2 = 512
NM2 = M // BM2
SB2 = S // BM2
SBF = S // BMF


def _stats_tile(z):
    rows = jax.lax.broadcasted_iota(jnp.int32, (8, 128), 0)
    cols = jax.lax.broadcasted_iota(jnp.int32, (8, 128), 1)
    acc = jnp.zeros((8, 128), jnp.float32)
    for h in range(H):
        zh = z[:, h * E:(h + 1) * E]
        s = jnp.sum(zh)
        q = jnp.sum(zh * zh)
        acc = acc + jnp.where((rows == h) & (cols == 0), s, 0.0)
        acc = acc + jnp.where((rows == h) & (cols == 1), q, 0.0)
    return acc.reshape(1, 8, 128)


def _softmax_tile(z, c, bm):
    z4 = z.reshape(bm, H, E)
    rowmax = jnp.max(z4, axis=2, keepdims=True)
    t = jnp.minimum(rowmax, c)
    e = jnp.where(z4 >= t, jnp.exp(z4 - rowmax), 0.0)
    den = jnp.sum(e, axis=2, keepdims=True)
    return (e / den).reshape(1, bm, H, E)


def _mm_stats_kernel(x_ref, w_ref, z_ref, st_ref):
    z = jax.lax.dot_general(
        x_ref[...], w_ref[...], (((1,), (1,)), ((), ())),
        preferred_element_type=jnp.float32)
    z_ref[...] = z
    st_ref[...] = _stats_tile(z)


def _mm_stats(x2, w):
    return pl.pallas_call(
        _mm_stats_kernel,
        grid=(NM1,),
        in_specs=[
            pl.BlockSpec((BM1, D), lambda m: (m, 0)),
            pl.BlockSpec((HE, D), lambda m: (0, 0)),
        ],
        out_specs=[
            pl.BlockSpec((BM1, HE), lambda m: (m, 0)),
            pl.BlockSpec((1, 8, 128), lambda m: (m, 0, 0)),
        ],
        out_shape=[
            jax.ShapeDtypeStruct((M, HE), jnp.float32),
            jax.ShapeDtypeStruct((NM1, 8, 128), jnp.float32),
        ],
    )(x2, w)


def _thresholds(st):
    s = st[:, :, 0].sum(axis=0)
    q = st[:, :, 1].sum(axis=0)
    n = float(M * E)
    mean = s / n
    var = q / n - mean * mean
    p = 1.0 - float(TOPK) / float(E)
    sigma = jnp.sqrt(2.0) * jax.scipy.special.erfinv(2.0 * p - 1.0)
    c = sigma * jnp.sqrt(var + EPS) + mean
    return jnp.broadcast_to(c[:, None], (H, 128))


def _mm_softmax_kernel(x_ref, w_ref, za_ref, c_ref, zb_ref, st_ref, o_ref):
    z = jax.lax.dot_general(
        x_ref[...], w_ref[...], (((1,), (1,)), ((), ())),
        preferred_element_type=jnp.float32)
    zb_ref[...] = z
    st_ref[...] = _stats_tile(z)
    c = c_ref[...][:, :1]
    o_ref[...] = _softmax_tile(za_ref[...], c, BMF)


def _mm_softmax(x2, w, za, c):
    return pl.pallas_call(
        _mm_softmax_kernel,
        grid=(NMF,),
        in_specs=[
            pl.BlockSpec((BMF, D), lambda m: (m, 0)),
            pl.BlockSpec((HE, D), lambda m: (0, 0)),
            pl.BlockSpec((BMF, HE), lambda m: (m, 0)),
            pl.BlockSpec((H, 128), lambda m: (0, 0)),
        ],
        out_specs=[
            pl.BlockSpec((BMF, HE), lambda m: (m, 0)),
            pl.BlockSpec((1, 8, 128), lambda m: (m, 0, 0)),
            pl.BlockSpec((1, BMF, H, E), lambda m: (m // SBF, m % SBF, 0, 0)),
        ],
        out_shape=[
            jax.ShapeDtypeStruct((M, HE), jnp.float32),
            jax.ShapeDtypeStruct((NMF, 8, 128), jnp.float32),
            jax.ShapeDtypeStruct((B, S, H, E), jnp.float32),
        ],
    )(x2, w, za, c)


def _softmax_kernel(z_ref, c_ref, o_ref):
    c = c_ref[...][:, :1]
    o_ref[...] = _softmax_tile(z_ref[...], c, BM2)


def _masked_softmax(z, c):
    return pl.pallas_call(
        _softmax_kernel,
        grid=(NM2,),
        in_specs=[
            pl.BlockSpec((BM2, HE), lambda m: (m, 0)),
            pl.BlockSpec((H, 128), lambda m: (0, 0)),
        ],
        out_specs=pl.BlockSpec(
            (1, BM2, H, E), lambda m: (m // SB2, m % SB2, 0, 0)),
        out_shape=jax.ShapeDtypeStruct((B, S, H, E), jnp.float32),
    )(z, c)


def kernel(x, W1, W2):
    x2 = x.reshape(M, D)
    z1, st1 = _mm_stats(x2, W1)
    c1 = _thresholds(st1)
    z2, st2, g1 = _mm_softmax(x2, W2, z1, c1)
    c2 = _thresholds(st2)
    g2 = _masked_softmax(z2, c2)
    return g1, g2


# speedup vs baseline: 1.5129x; 1.3342x over previous
import jax
import jax.numpy as jnp
from jax.experimental import pallas as pl

B, S, D = 4, 2048, 2048
H, E = 8, 512
TOPK = 8
EPS = 1e-5

M = B * S
HE = H * E
BM1 = 256
NM1 = M // BM1
BMF = 128
NMF = M // BMF
BM2 = 512
NM2 = M // BM2
SB2 = S // BM2
SBF = S // BMF


def _stats_tile(z):
    rows = jax.lax.broadcasted_iota(jnp.int32, (8, 128), 0)
    cols = jax.lax.broadcasted_iota(jnp.int32, (8, 128), 1)
    acc = jnp.zeros((8, 128), jnp.float32)
    for h in range(H):
        zh = z[:, h * E:(h + 1) * E]
        s = jnp.sum(zh)
        q = jnp.sum(zh * zh)
        acc = acc + jnp.where((rows == h) & (cols == 0), s, 0.0)
        acc = acc + jnp.where((rows == h) & (cols == 1), q, 0.0)
    return acc.reshape(1, 8, 128)


def _softmax_tile(z, c, bm):
    z4 = z.reshape(bm, H, E)
    rowmax = jnp.max(z4, axis=2, keepdims=True)
    t = jnp.minimum(rowmax, c)
    e = jnp.where(z4 >= t, jnp.exp(z4 - rowmax), 0.0)
    den = jnp.sum(e, axis=2, keepdims=True)
    return (e / den).reshape(1, bm, H, E)


def _mm_stats_kernel(x_ref, w_ref, z_ref, st_ref):
    z = jax.lax.dot_general(
        x_ref[...], w_ref[...], (((1,), (1,)), ((), ())),
        preferred_element_type=jnp.float32)
    z_ref[...] = z
    st_ref[...] = _stats_tile(z)


def _mm_stats(x2, w):
    return pl.pallas_call(
        _mm_stats_kernel,
        grid=(NM1,),
        in_specs=[
            pl.BlockSpec((BM1, D), lambda m: (m, 0)),
            pl.BlockSpec((HE, D), lambda m: (0, 0)),
        ],
        out_specs=[
            pl.BlockSpec((BM1, HE), lambda m: (m, 0)),
            pl.BlockSpec((1, 8, 128), lambda m: (m, 0, 0)),
        ],
        out_shape=[
            jax.ShapeDtypeStruct((M, HE), jnp.float32),
            jax.ShapeDtypeStruct((NM1, 8, 128), jnp.float32),
        ],
    )(x2, w)


def _thresholds(st):
    s = st[:, :, 0].sum(axis=0)
    q = st[:, :, 1].sum(axis=0)
    n = float(M * E)
    mean = s / n
    var = q / n - mean * mean
    p = 1.0 - float(TOPK) / float(E)
    sigma = jnp.sqrt(2.0) * jax.scipy.special.erfinv(2.0 * p - 1.0)
    c = sigma * jnp.sqrt(var + EPS) + mean
    return jnp.broadcast_to(c[:, None], (H, 128))


def _mm_softmax_kernel(x_ref, w_ref, za_ref, c_ref, zb_ref, st_ref, o_ref):
    z = jax.lax.dot_general(
        x_ref[...], w_ref[...], (((1,), (1,)), ((), ())),
        preferred_element_type=jnp.float32)
    zb_ref[...] = z
    st_ref[...] = _stats_tile(z)
    c = c_ref[...][:, :1]
    o_ref[...] = _softmax_tile(za_ref[...], c, BMF)


def _mm_softmax(x2, w, za, c):
    return pl.pallas_call(
        _mm_softmax_kernel,
        grid=(NMF,),
        in_specs=[
            pl.BlockSpec((BMF, D), lambda m: (m, 0)),
            pl.BlockSpec((HE, D), lambda m: (0, 0)),
            pl.BlockSpec((BMF, HE), lambda m: (m, 0)),
            pl.BlockSpec((H, 128), lambda m: (0, 0)),
        ],
        out_specs=[
            pl.BlockSpec((BMF, HE), lambda m: (m, 0)),
            pl.BlockSpec((1, 8, 128), lambda m: (m, 0, 0)),
            pl.BlockSpec((1, BMF, H, E), lambda m: (m // SBF, m % SBF, 0, 0)),
        ],
        out_shape=[
            jax.ShapeDtypeStruct((M, HE), jnp.float32),
            jax.ShapeDtypeStruct((NMF, 8, 128), jnp.float32),
            jax.ShapeDtypeStruct((B, S, H, E), jnp.float32),
        ],
    )(x2, w, za, c)


def _softmax_kernel(z_ref, c_ref, o_ref):
    c = c_ref[...][:, :1]
    o_ref[...] = _softmax_tile(z_ref[...], c, BM2)


def _masked_softmax(z, c):
    return pl.pallas_call(
        _softmax_kernel,
        grid=(NM2,),
        in_specs=[
            pl.BlockSpec((BM2, HE), lambda m: (m, 0)),
            pl.BlockSpec((H, 128), lambda m: (0, 0)),
        ],
        out_specs=pl.BlockSpec(
            (1, BM2, H, E), lambda m: (m // SB2, m % SB2, 0, 0)),
        out_shape=jax.ShapeDtypeStruct((B, S, H, E), jnp.float32),
    )(z, c)


def kernel(x, W1, W2):
    x2 = x.reshape(M, D)
    z1, st1 = _mm_stats(x2, W1)
    z2, st2 = _mm_stats(x2, W2)
    c1 = _thresholds(st1)
    c2 = _thresholds(st2)
    g1 = _masked_softmax(z1, c1)
    g2 = _masked_softmax(z2, c2)
    return g1, g2


# speedup vs baseline: 1.5379x; 1.0165x over previous
import jax
import jax.numpy as jnp
from jax.experimental import pallas as pl

B, S, D = 4, 2048, 2048
H, E = 8, 512
TOPK = 8
EPS = 1e-5

M = B * S
HE = H * E
BM1 = 256
NM1 = M // BM1
BMF = 128
NMF = M // BMF
BM2 = 512
NM2 = M // BM2
SB2 = S // BM2
SBF = S // BMF
BMS = 256
NMS = M // BMS
SBS = S // BMS


def _stats_tile(z):
    rows = jax.lax.broadcasted_iota(jnp.int32, (8, 128), 0)
    cols = jax.lax.broadcasted_iota(jnp.int32, (8, 128), 1)
    acc = jnp.zeros((8, 128), jnp.float32)
    for h in range(H):
        zh = z[:, h * E:(h + 1) * E]
        s = jnp.sum(zh)
        q = jnp.sum(zh * zh)
        acc = acc + jnp.where((rows == h) & (cols == 0), s, 0.0)
        acc = acc + jnp.where((rows == h) & (cols == 1), q, 0.0)
    return acc.reshape(1, 8, 128)


def _softmax_tile(z, c, bm):
    z4 = z.reshape(bm, H, E)
    rowmax = jnp.max(z4, axis=2, keepdims=True)
    t = jnp.minimum(rowmax, c)
    e = jnp.where(z4 >= t, jnp.exp(z4 - rowmax), 0.0)
    den = jnp.sum(e, axis=2, keepdims=True)
    return (e / den).reshape(1, bm, H, E)


def _mm_stats_kernel(x_ref, w_ref, z_ref, st_ref):
    z = jax.lax.dot_general(
        x_ref[...], w_ref[...], (((1,), (1,)), ((), ())),
        preferred_element_type=jnp.float32)
    z_ref[...] = z
    st_ref[...] = _stats_tile(z)


def _mm_stats(x2, w):
    return pl.pallas_call(
        _mm_stats_kernel,
        grid=(NM1,),
        in_specs=[
            pl.BlockSpec((BM1, D), lambda m: (m, 0)),
            pl.BlockSpec((HE, D), lambda m: (0, 0)),
        ],
        out_specs=[
            pl.BlockSpec((BM1, HE), lambda m: (m, 0)),
            pl.BlockSpec((1, 8, 128), lambda m: (m, 0, 0)),
        ],
        out_shape=[
            jax.ShapeDtypeStruct((M, HE), jnp.float32),
            jax.ShapeDtypeStruct((NM1, 8, 128), jnp.float32),
        ],
    )(x2, w)


def _thresholds(st):
    s = st[:, :, 0].sum(axis=0)
    q = st[:, :, 1].sum(axis=0)
    n = float(M * E)
    mean = s / n
    var = q / n - mean * mean
    p = 1.0 - float(TOPK) / float(E)
    sigma = jnp.sqrt(2.0) * jax.scipy.special.erfinv(2.0 * p - 1.0)
    c = sigma * jnp.sqrt(var + EPS) + mean
    return jnp.broadcast_to(c[:, None], (H, 128))


def _mm_softmax_kernel(x_ref, w_ref, za_ref, c_ref, zb_ref, st_ref, o_ref):
    z = jax.lax.dot_general(
        x_ref[...], w_ref[...], (((1,), (1,)), ((), ())),
        preferred_element_type=jnp.float32)
    zb_ref[...] = z
    st_ref[...] = _stats_tile(z)
    c = c_ref[...][:, :1]
    o_ref[...] = _softmax_tile(za_ref[...], c, BMF)


def _mm_softmax(x2, w, za, c):
    return pl.pallas_call(
        _mm_softmax_kernel,
        grid=(NMF,),
        in_specs=[
            pl.BlockSpec((BMF, D), lambda m: (m, 0)),
            pl.BlockSpec((HE, D), lambda m: (0, 0)),
            pl.BlockSpec((BMF, HE), lambda m: (m, 0)),
            pl.BlockSpec((H, 128), lambda m: (0, 0)),
        ],
        out_specs=[
            pl.BlockSpec((BMF, HE), lambda m: (m, 0)),
            pl.BlockSpec((1, 8, 128), lambda m: (m, 0, 0)),
            pl.BlockSpec((1, BMF, H, E), lambda m: (m // SBF, m % SBF, 0, 0)),
        ],
        out_shape=[
            jax.ShapeDtypeStruct((M, HE), jnp.float32),
            jax.ShapeDtypeStruct((NMF, 8, 128), jnp.float32),
            jax.ShapeDtypeStruct((B, S, H, E), jnp.float32),
        ],
    )(x2, w, za, c)


def _softmax2_kernel(z1_ref, z2_ref, c1_ref, c2_ref, o1_ref, o2_ref):
    o1_ref[...] = _softmax_tile(z1_ref[...], c1_ref[...][:, :1], BMS)
    o2_ref[...] = _softmax_tile(z2_ref[...], c2_ref[...][:, :1], BMS)


def _masked_softmax2(z1, z2, c1, c2):
    return pl.pallas_call(
        _softmax2_kernel,
        grid=(NMS,),
        in_specs=[
            pl.BlockSpec((BMS, HE), lambda m: (m, 0)),
            pl.BlockSpec((BMS, HE), lambda m: (m, 0)),
            pl.BlockSpec((H, 128), lambda m: (0, 0)),
            pl.BlockSpec((H, 128), lambda m: (0, 0)),
        ],
        out_specs=[
            pl.BlockSpec((1, BMS, H, E), lambda m: (m // SBS, m % SBS, 0, 0)),
            pl.BlockSpec((1, BMS, H, E), lambda m: (m // SBS, m % SBS, 0, 0)),
        ],
        out_shape=[
            jax.ShapeDtypeStruct((B, S, H, E), jnp.float32),
            jax.ShapeDtypeStruct((B, S, H, E), jnp.float32),
        ],
    )(z1, z2, c1, c2)


def _softmax_kernel(z_ref, c_ref, o_ref):
    c = c_ref[...][:, :1]
    o_ref[...] = _softmax_tile(z_ref[...], c, BM2)


def _masked_softmax(z, c):
    return pl.pallas_call(
        _softmax_kernel,
        grid=(NM2,),
        in_specs=[
            pl.BlockSpec((BM2, HE), lambda m: (m, 0)),
            pl.BlockSpec((H, 128), lambda m: (0, 0)),
        ],
        out_specs=pl.BlockSpec(
            (1, BM2, H, E), lambda m: (m // SB2, m % SB2, 0, 0)),
        out_shape=jax.ShapeDtypeStruct((B, S, H, E), jnp.float32),
    )(z, c)


def kernel(x, W1, W2):
    x2 = x.reshape(M, D)
    z1, st1 = _mm_stats(x2, W1)
    z2, st2 = _mm_stats(x2, W2)
    c1 = _thresholds(st1)
    c2 = _thresholds(st2)
    g1, g2 = _masked_softmax2(z1, z2, c1, c2)
    return g1, g2


# speedup vs baseline: 1.5758x; 1.0247x over previous
import jax
import jax.numpy as jnp
from jax.experimental import pallas as pl

B, S, D = 4, 2048, 2048
H, E = 8, 512
TOPK = 8
EPS = 1e-5

M = B * S
HE = H * E
BM1 = 512
NM1 = M // BM1
BMF = 128
NMF = M // BMF
BM2 = 512
NM2 = M // BM2
SB2 = S // BM2
SBF = S // BMF
BMS = 256
NMS = M // BMS
SBS = S // BMS


def _stats_tile(z):
    rows = jax.lax.broadcasted_iota(jnp.int32, (8, 128), 0)
    cols = jax.lax.broadcasted_iota(jnp.int32, (8, 128), 1)
    acc = jnp.zeros((8, 128), jnp.float32)
    for h in range(H):
        zh = z[:, h * E:(h + 1) * E]
        s = jnp.sum(zh)
        q = jnp.sum(zh * zh)
        acc = acc + jnp.where((rows == h) & (cols == 0), s, 0.0)
        acc = acc + jnp.where((rows == h) & (cols == 1), q, 0.0)
    return acc.reshape(1, 8, 128)


def _softmax_tile(z, c, bm):
    z4 = z.reshape(bm, H, E)
    rowmax = jnp.max(z4, axis=2, keepdims=True)
    t = jnp.minimum(rowmax, c)
    e = jnp.where(z4 >= t, jnp.exp(z4 - rowmax), 0.0)
    den = jnp.sum(e, axis=2, keepdims=True)
    return (e / den).reshape(1, bm, H, E)


def _mm_stats_kernel(x_ref, w_ref, z_ref, st_ref):
    z = jax.lax.dot_general(
        x_ref[...], w_ref[...], (((1,), (1,)), ((), ())),
        preferred_element_type=jnp.float32)
    z_ref[...] = z
    st_ref[...] = _stats_tile(z)


def _mm_stats(x2, w):
    return pl.pallas_call(
        _mm_stats_kernel,
        grid=(NM1,),
        in_specs=[
            pl.BlockSpec((BM1, D), lambda m: (m, 0)),
            pl.BlockSpec((HE, D), lambda m: (0, 0)),
        ],
        out_specs=[
            pl.BlockSpec((BM1, HE), lambda m: (m, 0)),
            pl.BlockSpec((1, 8, 128), lambda m: (m, 0, 0)),
        ],
        out_shape=[
            jax.ShapeDtypeStruct((M, HE), jnp.float32),
            jax.ShapeDtypeStruct((NM1, 8, 128), jnp.float32),
        ],
    )(x2, w)


def _thresholds(st):
    s = st[:, :, 0].sum(axis=0)
    q = st[:, :, 1].sum(axis=0)
    n = float(M * E)
    mean = s / n
    var = q / n - mean * mean
    p = 1.0 - float(TOPK) / float(E)
    sigma = jnp.sqrt(2.0) * jax.scipy.special.erfinv(2.0 * p - 1.0)
    c = sigma * jnp.sqrt(var + EPS) + mean
    return jnp.broadcast_to(c[:, None], (H, 128))


def _mm_softmax_kernel(x_ref, w_ref, za_ref, c_ref, zb_ref, st_ref, o_ref):
    z = jax.lax.dot_general(
        x_ref[...], w_ref[...], (((1,), (1,)), ((), ())),
        preferred_element_type=jnp.float32)
    zb_ref[...] = z
    st_ref[...] = _stats_tile(z)
    c = c_ref[...][:, :1]
    o_ref[...] = _softmax_tile(za_ref[...], c, BMF)


def _mm_softmax(x2, w, za, c):
    return pl.pallas_call(
        _mm_softmax_kernel,
        grid=(NMF,),
        in_specs=[
            pl.BlockSpec((BMF, D), lambda m: (m, 0)),
            pl.BlockSpec((HE, D), lambda m: (0, 0)),
            pl.BlockSpec((BMF, HE), lambda m: (m, 0)),
            pl.BlockSpec((H, 128), lambda m: (0, 0)),
        ],
        out_specs=[
            pl.BlockSpec((BMF, HE), lambda m: (m, 0)),
            pl.BlockSpec((1, 8, 128), lambda m: (m, 0, 0)),
            pl.BlockSpec((1, BMF, H, E), lambda m: (m // SBF, m % SBF, 0, 0)),
        ],
        out_shape=[
            jax.ShapeDtypeStruct((M, HE), jnp.float32),
            jax.ShapeDtypeStruct((NMF, 8, 128), jnp.float32),
            jax.ShapeDtypeStruct((B, S, H, E), jnp.float32),
        ],
    )(x2, w, za, c)


def _softmax2_kernel(z1_ref, z2_ref, c1_ref, c2_ref, o1_ref, o2_ref):
    o1_ref[...] = _softmax_tile(z1_ref[...], c1_ref[...][:, :1], BMS)
    o2_ref[...] = _softmax_tile(z2_ref[...], c2_ref[...][:, :1], BMS)


def _masked_softmax2(z1, z2, c1, c2):
    return pl.pallas_call(
        _softmax2_kernel,
        grid=(NMS,),
        in_specs=[
            pl.BlockSpec((BMS, HE), lambda m: (m, 0)),
            pl.BlockSpec((BMS, HE), lambda m: (m, 0)),
            pl.BlockSpec((H, 128), lambda m: (0, 0)),
            pl.BlockSpec((H, 128), lambda m: (0, 0)),
        ],
        out_specs=[
            pl.BlockSpec((1, BMS, H, E), lambda m: (m // SBS, m % SBS, 0, 0)),
            pl.BlockSpec((1, BMS, H, E), lambda m: (m // SBS, m % SBS, 0, 0)),
        ],
        out_shape=[
            jax.ShapeDtypeStruct((B, S, H, E), jnp.float32),
            jax.ShapeDtypeStruct((B, S, H, E), jnp.float32),
        ],
    )(z1, z2, c1, c2)


def _softmax_kernel(z_ref, c_ref, o_ref):
    c = c_ref[...][:, :1]
    o_ref[...] = _softmax_tile(z_ref[...], c, BM2)


def _masked_softmax(z, c):
    return pl.pallas_call(
        _softmax_kernel,
        grid=(NM2,),
        in_specs=[
            pl.BlockSpec((BM2, HE), lambda m: (m, 0)),
            pl.BlockSpec((H, 128), lambda m: (0, 0)),
        ],
        out_specs=pl.BlockSpec(
            (1, BM2, H, E), lambda m: (m // SB2, m % SB2, 0, 0)),
        out_shape=jax.ShapeDtypeStruct((B, S, H, E), jnp.float32),
    )(z, c)


def kernel(x, W1, W2):
    x2 = x.reshape(M, D)
    z1, st1 = _mm_stats(x2, W1)
    z2, st2 = _mm_stats(x2, W2)
    c1 = _thresholds(st1)
    c2 = _thresholds(st2)
    g1, g2 = _masked_softmax2(z1, z2, c1, c2)
    return g1, g2


# speedup vs baseline: 1.5867x; 1.0069x over previous
import math

import jax
import jax.numpy as jnp
from jax.experimental import pallas as pl

B, S, D = 4, 2048, 2048
H, E = 8, 512
TOPK = 8
EPS = 1e-5

M = B * S
HE = H * E
BM1 = 512
NM1 = M // BM1
BMF = 128
NMF = M // BMF
BM2 = 512
NM2 = M // BM2
SB2 = S // BM2
SBF = S // BMF
BMS = 256
NMS = M // BMS
SBS = S // BMS


def _norm_ppf(p):
    lo, hi = 0.0, 10.0
    target = 2.0 * p - 1.0
    for _ in range(200):
        mid = (lo + hi) / 2.0
        if math.erf(mid / math.sqrt(2.0)) < target:
            lo = mid
        else:
            hi = mid
    return (lo + hi) / 2.0


SIGMA = _norm_ppf(1.0 - float(TOPK) / float(E))


def _stats_tile(z):
    rows = jax.lax.broadcasted_iota(jnp.int32, (8, 128), 0)
    cols = jax.lax.broadcasted_iota(jnp.int32, (8, 128), 1)
    acc = jnp.zeros((8, 128), jnp.float32)
    for h in range(H):
        zh = z[:, h * E:(h + 1) * E]
        s = jnp.sum(zh)
        q = jnp.sum(zh * zh)
        acc = acc + jnp.where((rows == h) & (cols == 0), s, 0.0)
        acc = acc + jnp.where((rows == h) & (cols == 1), q, 0.0)
    return acc.reshape(1, 8, 128)


def _softmax_tile(z, c, bm):
    z4 = z.reshape(bm, H, E)
    rowmax = jnp.max(z4, axis=2, keepdims=True)
    t = jnp.minimum(rowmax, c)
    e = jnp.where(z4 >= t, jnp.exp(z4 - rowmax), 0.0)
    den = jnp.sum(e, axis=2, keepdims=True)
    return (e / den).reshape(1, bm, H, E)


def _mm_stats_kernel(x_ref, w_ref, z_ref, st_ref):
    z = jax.lax.dot_general(
        x_ref[...], w_ref[...], (((1,), (1,)), ((), ())),
        preferred_element_type=jnp.float32)
    z_ref[...] = z
    st_ref[...] = _stats_tile(z)


def _mm_stats(x2, w):
    return pl.pallas_call(
        _mm_stats_kernel,
        grid=(NM1,),
        in_specs=[
            pl.BlockSpec((BM1, D), lambda m: (m, 0)),
            pl.BlockSpec((HE, D), lambda m: (0, 0)),
        ],
        out_specs=[
            pl.BlockSpec((BM1, HE), lambda m: (m, 0)),
            pl.BlockSpec((1, 8, 128), lambda m: (m, 0, 0)),
        ],
        out_shape=[
            jax.ShapeDtypeStruct((M, HE), jnp.float32),
            jax.ShapeDtypeStruct((NM1, 8, 128), jnp.float32),
        ],
    )(x2, w)


def _thresholds(st):
    s = st[:, :, 0].sum(axis=0)
    q = st[:, :, 1].sum(axis=0)
    n = float(M * E)
    mean = s / n
    var = q / n - mean * mean
    p = 1.0 - float(TOPK) / float(E)
    sigma = jnp.sqrt(2.0) * jax.scipy.special.erfinv(2.0 * p - 1.0)
    c = sigma * jnp.sqrt(var + EPS) + mean
    return jnp.broadcast_to(c[:, None], (H, 128))


def _mm_softmax_kernel(x_ref, w_ref, za_ref, c_ref, zb_ref, st_ref, o_ref):
    z = jax.lax.dot_general(
        x_ref[...], w_ref[...], (((1,), (1,)), ((), ())),
        preferred_element_type=jnp.float32)
    zb_ref[...] = z
    st_ref[...] = _stats_tile(z)
    c = c_ref[...][:, :1]
    o_ref[...] = _softmax_tile(za_ref[...], c, BMF)


def _mm_softmax(x2, w, za, c):
    return pl.pallas_call(
        _mm_softmax_kernel,
        grid=(NMF,),
        in_specs=[
            pl.BlockSpec((BMF, D), lambda m: (m, 0)),
            pl.BlockSpec((HE, D), lambda m: (0, 0)),
            pl.BlockSpec((BMF, HE), lambda m: (m, 0)),
            pl.BlockSpec((H, 128), lambda m: (0, 0)),
        ],
        out_specs=[
            pl.BlockSpec((BMF, HE), lambda m: (m, 0)),
            pl.BlockSpec((1, 8, 128), lambda m: (m, 0, 0)),
            pl.BlockSpec((1, BMF, H, E), lambda m: (m // SBF, m % SBF, 0, 0)),
        ],
        out_shape=[
            jax.ShapeDtypeStruct((M, HE), jnp.float32),
            jax.ShapeDtypeStruct((NMF, 8, 128), jnp.float32),
            jax.ShapeDtypeStruct((B, S, H, E), jnp.float32),
        ],
    )(x2, w, za, c)


def _cvec(st):
    t = jnp.sum(st, axis=0)
    n = float(M * E)
    mean = t[:, 0:1] / n
    var = t[:, 1:2] / n - mean * mean
    return SIGMA * jnp.sqrt(var + EPS) + mean


def _softmax2_kernel(st1_ref, st2_ref, z1_ref, z2_ref, o1_ref, o2_ref):
    o1_ref[...] = _softmax_tile(z1_ref[...], _cvec(st1_ref[...]), BMS)
    o2_ref[...] = _softmax_tile(z2_ref[...], _cvec(st2_ref[...]), BMS)


def _masked_softmax2(st1, st2, z1, z2):
    return pl.pallas_call(
        _softmax2_kernel,
        grid=(NMS,),
        in_specs=[
            pl.BlockSpec((NM1, 8, 128), lambda m: (0, 0, 0)),
            pl.BlockSpec((NM1, 8, 128), lambda m: (0, 0, 0)),
            pl.BlockSpec((BMS, HE), lambda m: (m, 0)),
            pl.BlockSpec((BMS, HE), lambda m: (m, 0)),
        ],
        out_specs=[
            pl.BlockSpec((1, BMS, H, E), lambda m: (m // SBS, m % SBS, 0, 0)),
            pl.BlockSpec((1, BMS, H, E), lambda m: (m // SBS, m % SBS, 0, 0)),
        ],
        out_shape=[
            jax.ShapeDtypeStruct((B, S, H, E), jnp.float32),
            jax.ShapeDtypeStruct((B, S, H, E), jnp.float32),
        ],
    )(st1, st2, z1, z2)


def _softmax_kernel(z_ref, c_ref, o_ref):
    c = c_ref[...][:, :1]
    o_ref[...] = _softmax_tile(z_ref[...], c, BM2)


def _masked_softmax(z, c):
    return pl.pallas_call(
        _softmax_kernel,
        grid=(NM2,),
        in_specs=[
            pl.BlockSpec((BM2, HE), lambda m: (m, 0)),
            pl.BlockSpec((H, 128), lambda m: (0, 0)),
        ],
        out_specs=pl.BlockSpec(
            (1, BM2, H, E), lambda m: (m // SB2, m % SB2, 0, 0)),
        out_shape=jax.ShapeDtypeStruct((B, S, H, E), jnp.float32),
    )(z, c)


def kernel(x, W1, W2):
    x2 = x.reshape(M, D)
    z1, st1 = _mm_stats(x2, W1)
    z2, st2 = _mm_stats(x2, W2)
    g1, g2 = _masked_softmax2(st1, st2, z1, z2)
    return g1, g2


# speedup vs baseline: 1.5871x; 1.0002x over previous
import math

import jax
import jax.numpy as jnp
from jax.experimental import pallas as pl

B, S, D = 4, 2048, 2048
H, E = 8, 512
TOPK = 8
EPS = 1e-5

M = B * S
HE = H * E
BM1 = 512
NM1 = M // BM1
BMS = 256
NMS = M // BMS
SBS = S // BMS


def _norm_ppf(p):
    lo, hi = 0.0, 10.0
    target = 2.0 * p - 1.0
    for _ in range(200):
        mid = (lo + hi) / 2.0
        if math.erf(mid / math.sqrt(2.0)) < target:
            lo = mid
        else:
            hi = mid
    return (lo + hi) / 2.0


SIGMA = _norm_ppf(1.0 - float(TOPK) / float(E))


def _stats_tile(z):
    rows = jax.lax.broadcasted_iota(jnp.int32, (8, 128), 0)
    cols = jax.lax.broadcasted_iota(jnp.int32, (8, 128), 1)
    acc = jnp.zeros((8, 128), jnp.float32)
    for h in range(H):
        zh = z[:, h * E:(h + 1) * E]
        s = jnp.sum(zh)
        q = jnp.sum(zh * zh)
        acc = acc + jnp.where((rows == h) & (cols == 0), s, 0.0)
        acc = acc + jnp.where((rows == h) & (cols == 1), q, 0.0)
    return acc.reshape(1, 8, 128)


def _mm_stats_kernel(x_ref, w_ref, z_ref, st_ref):
    z = jax.lax.dot_general(
        x_ref[...], w_ref[...], (((1,), (1,)), ((), ())),
        preferred_element_type=jnp.float32)
    z_ref[...] = z
    st_ref[...] = _stats_tile(z)


def _mm_stats(x2, w):
    return pl.pallas_call(
        _mm_stats_kernel,
        grid=(NM1,),
        in_specs=[
            pl.BlockSpec((BM1, D), lambda m: (m, 0)),
            pl.BlockSpec((HE, D), lambda m: (0, 0)),
        ],
        out_specs=[
            pl.BlockSpec((BM1, HE), lambda m: (m, 0)),
            pl.BlockSpec((1, 8, 128), lambda m: (m, 0, 0)),
        ],
        out_shape=[
            jax.ShapeDtypeStruct((M, HE), jnp.float32),
            jax.ShapeDtypeStruct((NM1, 8, 128), jnp.float32),
        ],
    )(x2, w)


def _cvec(st):
    t = jnp.sum(st, axis=0)
    n = float(M * E)
    mean = t[:, 0:1] / n
    var = t[:, 1:2] / n - mean * mean
    return SIGMA * jnp.sqrt(var + EPS) + mean


def _softmax_tile(z, c, bm):
    z4 = z.reshape(bm, H, E)
    rowmax = jnp.max(z4, axis=2, keepdims=True)
    t = jnp.minimum(rowmax, c)
    e = jnp.where(z4 >= t, jnp.exp(z4 - rowmax), 0.0)
    den = jnp.sum(e, axis=2, keepdims=True)
    return (e / den).reshape(1, bm, H, E)


def _softmax2_kernel(st1_ref, st2_ref, z1_ref, z2_ref, o1_ref, o2_ref):
    o1_ref[...] = _softmax_tile(z1_ref[...], _cvec(st1_ref[...]), BMS)
    o2_ref[...] = _softmax_tile(z2_ref[...], _cvec(st2_ref[...]), BMS)


def _masked_softmax2(st1, st2, z1, z2):
    return pl.pallas_call(
        _softmax2_kernel,
        grid=(NMS,),
        in_specs=[
            pl.BlockSpec((NM1, 8, 128), lambda m: (0, 0, 0)),
            pl.BlockSpec((NM1, 8, 128), lambda m: (0, 0, 0)),
            pl.BlockSpec((BMS, HE), lambda m: (m, 0)),
            pl.BlockSpec((BMS, HE), lambda m: (m, 0)),
        ],
        out_specs=[
            pl.BlockSpec((1, BMS, H, E), lambda m: (m // SBS, m % SBS, 0, 0)),
            pl.BlockSpec((1, BMS, H, E), lambda m: (m // SBS, m % SBS, 0, 0)),
        ],
        out_shape=[
            jax.ShapeDtypeStruct((B, S, H, E), jnp.float32),
            jax.ShapeDtypeStruct((B, S, H, E), jnp.float32),
        ],
    )(st1, st2, z1, z2)


def kernel(x, W1, W2):
    x2 = x.reshape(M, D)
    z1, st1 = _mm_stats(x2, W1)
    z2, st2 = _mm_stats(x2, W2)
    g1, g2 = _masked_softmax2(st1, st2, z1, z2)
    return g1, g2
